# baseline (speedup 1.0000x reference)
import functools

import jax
import jax.numpy as jnp
from jax import lax
from jax.experimental import pallas as pl
from jax.experimental.pallas import tpu as pltpu
from jax.experimental.pallas import tpu_sc as plsc

NC = 2
NS = 16
EB = 128
CW = 128
K = CW


def _sc_mesh():
    return plsc.VectorSubcoreMesh(
        core_axis_name="c", subcore_axis_name="s", num_cores=NC, num_subcores=NS
    )


DEGW = 128


def _sc_degree(dst3, ones_h, zeros_h, npad, nblk):
    rpt = npad // NS
    half = nblk // NC

    @functools.partial(
        pl.kernel,
        out_type=[jax.ShapeDtypeStruct((npad, DEGW), jnp.float32) for _ in range(NC)],
        mesh=_sc_mesh(),
        scratch_types=[
            pltpu.VMEM_SHARED((npad, DEGW), jnp.float32),
            pltpu.VMEM((nblk // NC, EB), jnp.int32),
            pltpu.VMEM((EB, DEGW), jnp.float32),
            pltpu.SemaphoreType.DMA,
            pltpu.SemaphoreType.DMA,
        ],
    )
    def deg_kernel(dst_ha, dst_hb, ones_hbm, zeros_hbm, out0, out1, acc, idx_d,
                   ones_v, s0, s1):
        cid = lax.axis_index("c")
        sid = lax.axis_index("s")
        pltpu.sync_copy(zeros_hbm, acc.at[pl.ds(sid * rpt, rpt)])
        pltpu.sync_copy(ones_hbm, ones_v)

        @pl.when(cid == 0)
        def _():
            pltpu.sync_copy(dst_ha.at[sid], idx_d)

        @pl.when(cid == 1)
        def _():
            pltpu.sync_copy(dst_hb.at[sid], idx_d)

        plsc.subcore_barrier()

        def blk(i, carry):
            pltpu.async_copy(ones_v, acc.at[idx_d.at[2 * i]], s0, add=True)
            pltpu.async_copy(ones_v, acc.at[idx_d.at[2 * i + 1]], s1, add=True)
            pltpu.make_async_copy(ones_v, acc.at[idx_d.at[2 * i]], s0).wait()
            pltpu.make_async_copy(ones_v, acc.at[idx_d.at[2 * i + 1]], s1).wait()
            return carry

        lax.fori_loop(0, half // 2, blk, 0)
        plsc.subcore_barrier()

        @pl.when(cid == 0)
        def _():
            pltpu.sync_copy(
                acc.at[pl.ds(sid * rpt, rpt)], out0.at[pl.ds(sid * rpt, rpt)]
            )

        @pl.when(cid == 1)
        def _():
            pltpu.sync_copy(
                acc.at[pl.ds(sid * rpt, rpt)], out1.at[pl.ds(sid * rpt, rpt)]
            )

    return deg_kernel(dst3[:, :half], dst3[:, half:], ones_h, zeros_h)


def _sc_spmm(g_chunks, src3, dst3, zeros_h, npad, nblk):
    nch = len(g_chunks)
    C = g_chunks[0].shape[1]
    rpt = npad // NS

    NBUF = 2
    NPH = 5
    HB = nblk // NPH

    @functools.partial(
        pl.kernel,
        out_type=[jax.ShapeDtypeStruct((npad, C), jnp.float32) for _ in range(nch)],
        mesh=_sc_mesh(),
        scratch_types=[
            pltpu.VMEM_SHARED((npad, C), jnp.float32),
            pltpu.VMEM((HB, EB), jnp.int32),
            pltpu.VMEM((HB, EB), jnp.int32),
        ]
        + [pltpu.VMEM((EB, C), jnp.float32) for _ in range(NBUF)]
        + [pltpu.SemaphoreType.DMA for _ in range(2 * NBUF)],
    )
    def spmm_kernel(src_h, dst_h, zeros_hbm, *rest):
        gs = rest[:nch]
        outs = rest[nch : 2 * nch]
        scr = rest[2 * nch :]
        acc, idx_s, idx_d = scr[:3]
        rows = scr[3 : 3 + NBUF]
        gsem = scr[3 + NBUF : 3 + 2 * NBUF]
        ssem = scr[3 + 2 * NBUF :]
        cid = lax.axis_index("c")
        sid = lax.axis_index("s")
        for ci in range(nch):

            @pl.when(cid == (ci % NC))
            def _(ci=ci):
                g = gs[ci]
                o = outs[ci]
                pltpu.sync_copy(zeros_hbm, acc.at[pl.ds(sid * rpt, rpt)])
                plsc.subcore_barrier()

                for p in range(NPH):

                    def stage(p=p):
                        pltpu.sync_copy(src_h.at[sid, pl.ds(p * HB, HB)], idx_s)
                        pltpu.sync_copy(dst_h.at[sid, pl.ds(p * HB, HB)], idx_d)

                        def step(i, carry):
                            for b in range(NBUF):
                                j = i * NBUF + b
                                pltpu.async_copy(g.at[idx_s.at[j]], rows[b], gsem[b])
                            for b in range(NBUF):
                                j = i * NBUF + b
                                pltpu.make_async_copy(
                                    g.at[idx_s.at[j]], rows[b], gsem[b]
                                ).wait()
                                pltpu.async_copy(
                                    rows[b], acc.at[idx_d.at[j]], ssem[b], add=True
                                )
                            for b in range(NBUF):
                                j = i * NBUF + b
                                pltpu.make_async_copy(
                                    rows[b], acc.at[idx_d.at[j]], ssem[b]
                                ).wait()
                            return carry

                        lax.fori_loop(0, HB // NBUF, step, 0)

                    stage()
                plsc.subcore_barrier()
                pltpu.sync_copy(
                    acc.at[pl.ds(sid * rpt, rpt)], o.at[pl.ds(sid * rpt, rpt)]
                )
                plsc.subcore_barrier()

    return list(spmm_kernel(src3, dst3, zeros_h, *g_chunks))


def _dinv_col(deg_refs, n):
    d0, d1 = deg_refs
    return lax.rsqrt(d0[0:n, 0:1] + d1[0:n, 0:1] + 1.0)


def _tc_prep(x, w_f1, w_b1, deg_t, n, d_hid):
    nch = (2 * d_hid) // K

    def body(x_ref, wf_ref, wb_ref, deg0, deg1, *outs):
        dinv = _dinv_col((deg0, deg1), n)
        w = jnp.concatenate([wf_ref[...], wb_ref[...]], axis=1)
        g = jnp.dot(x_ref[...], w, preferred_element_type=jnp.float32) * dinv
        for i, o in enumerate(outs):
            o[...] = g[:, i * K : (i + 1) * K]

    return pl.pallas_call(
        body,
        out_shape=[jax.ShapeDtypeStruct((n, K), jnp.float32) for _ in range(nch)],
    )(x, w_f1, w_b1, *deg_t)


def _tc_mid(s1, g1, deg_t, b_f1, b_b1, w_f2, w_b2, n, d_hid):
    nch_in = len(g1)
    RB = 2000

    def body(*refs):
        s_refs = refs[:nch_in]
        g_refs = refs[nch_in : 2 * nch_in]
        deg0, deg1, bf_ref, bb_ref, wf_ref, wb_ref, o0, o1 = refs[2 * nch_in :]
        dinv = lax.rsqrt(deg0[:, 0:1] + deg1[:, 0:1] + 1.0)
        b1 = jnp.concatenate([bf_ref[...], bb_ref[...]])
        h1 = jnp.concatenate(
            [s_refs[i][...] + g_refs[i][...] for i in range(nch_in)], axis=1
        )
        h1 = dinv * h1 + b1[None, :]
        g2f = jnp.dot(h1[:, :d_hid], wf_ref[...], preferred_element_type=jnp.float32)
        g2b = jnp.dot(h1[:, d_hid:], wb_ref[...], preferred_element_type=jnp.float32)
        o0[...] = g2f * dinv
        o1[...] = g2b * dinv

    row_spec = pl.BlockSpec((RB, CW), lambda i: (i, 0))
    vec_spec = pl.BlockSpec((d_hid,), lambda i: (0,))
    mat_spec = pl.BlockSpec((d_hid, CW), lambda i: (0, 0))
    return pl.pallas_call(
        body,
        grid=(n // RB,),
        in_specs=[row_spec] * (2 * nch_in + 2) + [vec_spec] * 2 + [mat_spec] * 2,
        out_specs=[row_spec] * 2,
        out_shape=[jax.ShapeDtypeStruct((n, K), jnp.float32) for _ in range(2)],
    )(*s1, *g1, *deg_t, b_f1, b_b1, w_f2, w_b2)


def _tc_final(s2, g2, deg_t, b_f2, b_b2, gamma, beta, w_lin, b_lin, n, d_out):
    def body(s0, s1, g0, g1, deg0, deg1, bf_ref, bb_ref, gam_ref, bet_ref,
             wl_ref, bl_ref, out_ref):
        dinv = _dinv_col((deg0, deg1), n)
        b2 = jnp.concatenate([bf_ref[...], bb_ref[...]])
        h2 = jnp.concatenate(
            [s0[0:n, :] + g0[...], s1[0:n, :] + g1[...]], axis=1
        )
        h2 = dinv * h2 + b2[None, :]
        mean = jnp.mean(h2, axis=0, keepdims=True)
        cen = h2 - mean
        var = jnp.mean(cen * cen, axis=0, keepdims=True)
        hn = cen * lax.rsqrt(var + 1e-5)
        hn = hn * gam_ref[...][None, :] + bet_ref[...][None, :]
        out_ref[...] = (
            jnp.dot(hn, wl_ref[...], preferred_element_type=jnp.float32)
            + bl_ref[...][None, :]
        )

    return pl.pallas_call(
        body,
        out_shape=jax.ShapeDtypeStruct((n, d_out), jnp.float32),
    )(*s2, *g2, *deg_t, b_f2, b_b2, gamma, beta, w_lin, b_lin)


def kernel(x, edge_index, W_f1, b_f1, W_f2, b_f2, W_b1, b_b1, W_b2, b_b2,
           gamma, beta, W_lin, b_lin):
    n, _ = x.shape
    d_hid = W_f1.shape[1]
    d_out = W_f2.shape[1]
    e = edge_index.shape[1]

    ept = NS * EB
    nblk = ((e + ept - 1) // ept + 39) // 40 * 40
    epad = nblk * ept
    npad = (n // (NS * 8) + 1) * (NS * 8)

    src = edge_index[0].astype(jnp.int32)
    dst = edge_index[1].astype(jnp.int32)
    pad = epad - e
    src3 = jnp.concatenate([src, jnp.zeros((pad,), jnp.int32)])
    dst3 = jnp.concatenate([dst, jnp.full((pad,), n, jnp.int32)])
    src3 = src3.reshape(NS, nblk, EB)
    dst3 = dst3.reshape(NS, nblk, EB)

    ones_h = jnp.ones((EB, DEGW), jnp.float32)
    zeros16 = jnp.zeros((npad // NS, DEGW), jnp.float32)
    zerosK = jnp.zeros((npad // NS, CW), jnp.float32)

    deg_t = _sc_degree(dst3, ones_h, zeros16, npad, nblk)
    g1 = _tc_prep(x, W_f1, W_b1, deg_t, n, d_hid)
    s1 = _sc_spmm(g1, src3, dst3, zerosK, npad, nblk)
    g2 = _tc_mid(s1, g1, deg_t, b_f1, b_b1, W_f2, W_b2, n, d_hid)
    s2 = _sc_spmm(g2, src3, dst3, zerosK, npad, nblk)
    return _tc_final(s2, g2, deg_t, b_f2, b_b2, gamma, beta, W_lin, b_lin, n, d_out)


# speedup vs baseline: 1.8787x; 1.8787x over previous
import functools

import jax
import jax.numpy as jnp
from jax import lax
from jax.experimental import pallas as pl
from jax.experimental.pallas import tpu as pltpu
from jax.experimental.pallas import tpu_sc as plsc

NC = 2
NS = 16
EB = 128
CW = 128
K = CW


def _sc_mesh():
    return plsc.VectorSubcoreMesh(
        core_axis_name="c", subcore_axis_name="s", num_cores=NC, num_subcores=NS
    )


DEGW = 128


def _sc_degree(dst3, ones_h, zeros_h, npad, nblk):
    rpt = npad // NS
    half = nblk // NC

    @functools.partial(
        pl.kernel,
        out_type=[jax.ShapeDtypeStruct((npad, DEGW), jnp.float32) for _ in range(NC)],
        mesh=_sc_mesh(),
        scratch_types=[
            pltpu.VMEM_SHARED((npad, DEGW), jnp.float32),
            pltpu.VMEM((nblk // NC, EB), jnp.int32),
            pltpu.VMEM((EB, DEGW), jnp.float32),
            pltpu.SemaphoreType.DMA,
            pltpu.SemaphoreType.DMA,
        ],
    )
    def deg_kernel(dst_ha, dst_hb, ones_hbm, zeros_hbm, out0, out1, acc, idx_d,
                   ones_v, s0, s1):
        cid = lax.axis_index("c")
        sid = lax.axis_index("s")
        pltpu.sync_copy(zeros_hbm, acc.at[pl.ds(sid * rpt, rpt)])
        pltpu.sync_copy(ones_hbm, ones_v)

        @pl.when(cid == 0)
        def _():
            pltpu.sync_copy(dst_ha.at[sid], idx_d)

        @pl.when(cid == 1)
        def _():
            pltpu.sync_copy(dst_hb.at[sid], idx_d)

        plsc.subcore_barrier()

        def blk(i, carry):
            pltpu.async_copy(ones_v, acc.at[idx_d.at[2 * i]], s0, add=True)
            pltpu.async_copy(ones_v, acc.at[idx_d.at[2 * i + 1]], s1, add=True)
            pltpu.make_async_copy(ones_v, acc.at[idx_d.at[2 * i]], s0).wait()
            pltpu.make_async_copy(ones_v, acc.at[idx_d.at[2 * i + 1]], s1).wait()
            return carry

        lax.fori_loop(0, half // 2, blk, 0)
        plsc.subcore_barrier()

        @pl.when(cid == 0)
        def _():
            pltpu.sync_copy(
                acc.at[pl.ds(sid * rpt, rpt)], out0.at[pl.ds(sid * rpt, rpt)]
            )

        @pl.when(cid == 1)
        def _():
            pltpu.sync_copy(
                acc.at[pl.ds(sid * rpt, rpt)], out1.at[pl.ds(sid * rpt, rpt)]
            )

    return deg_kernel(dst3[:, :half], dst3[:, half:], ones_h, zeros_h)


def _sc_spmm(g_chunks, src3, dst3, zeros_h, npad, nblk):
    nch = len(g_chunks)
    C = g_chunks[0].shape[1]
    rpt = npad // NS

    @functools.partial(
        pl.kernel,
        out_type=[jax.ShapeDtypeStruct((npad, C), jnp.float32) for _ in range(nch)],
        mesh=_sc_mesh(),
        scratch_types=[
            pltpu.VMEM_SHARED((npad, C), jnp.float32),
            pltpu.VMEM((nblk, EB), jnp.int32),
            pltpu.VMEM((nblk, EB), jnp.int32),
            pltpu.VMEM((EB, C), jnp.float32),
            pltpu.SemaphoreType.DMA,
        ],
    )
    def spmm_kernel(src_h, dst_h, zeros_hbm, *rest):
        gs = rest[:nch]
        outs = rest[nch : 2 * nch]
        acc, idx_s, idx_d, rows, sem = rest[2 * nch :]
        cid = lax.axis_index("c")
        sid = lax.axis_index("s")
        pltpu.sync_copy(src_h.at[sid], idx_s)
        pltpu.sync_copy(dst_h.at[sid], idx_d)
        for ci in range(nch):

            @pl.when(cid == (ci % NC))
            def _(ci=ci):
                g = gs[ci]
                o = outs[ci]
                pltpu.sync_copy(zeros_hbm, acc.at[pl.ds(sid * rpt, rpt)])
                plsc.subcore_barrier()

                def blk(j, carry):
                    pltpu.async_copy(g.at[idx_s.at[j]], rows, sem).wait()
                    pltpu.sync_copy(rows, acc.at[idx_d.at[j]], add=True)
                    return carry

                lax.fori_loop(0, nblk, blk, 0)
                plsc.subcore_barrier()
                pltpu.sync_copy(
                    acc.at[pl.ds(sid * rpt, rpt)], o.at[pl.ds(sid * rpt, rpt)]
                )
                plsc.subcore_barrier()

    return list(spmm_kernel(src3, dst3, zeros_h, *g_chunks))


def _dinv_col(deg_refs, n):
    d0, d1 = deg_refs
    return lax.rsqrt(d0[0:n, 0:1] + d1[0:n, 0:1] + 1.0)


def _tc_prep(x, w_f1, w_b1, deg_t, n, d_hid):
    nch = (2 * d_hid) // K

    def body(x_ref, wf_ref, wb_ref, deg0, deg1, *outs):
        dinv = _dinv_col((deg0, deg1), n)
        w = jnp.concatenate([wf_ref[...], wb_ref[...]], axis=1)
        g = jnp.dot(x_ref[...], w, preferred_element_type=jnp.float32) * dinv
        for i, o in enumerate(outs):
            o[...] = g[:, i * K : (i + 1) * K]

    return pl.pallas_call(
        body,
        out_shape=[jax.ShapeDtypeStruct((n, K), jnp.float32) for _ in range(nch)],
    )(x, w_f1, w_b1, *deg_t)


def _tc_mid(s1, g1, deg_t, b_f1, b_b1, w_f2, w_b2, n, d_hid):
    nch_in = len(g1)
    RB = 2000

    def body(*refs):
        s_refs = refs[:nch_in]
        g_refs = refs[nch_in : 2 * nch_in]
        deg0, deg1, bf_ref, bb_ref, wf_ref, wb_ref, o0, o1 = refs[2 * nch_in :]
        dinv = lax.rsqrt(deg0[:, 0:1] + deg1[:, 0:1] + 1.0)
        b1 = jnp.concatenate([bf_ref[...], bb_ref[...]])
        h1 = jnp.concatenate(
            [s_refs[i][...] + g_refs[i][...] for i in range(nch_in)], axis=1
        )
        h1 = dinv * h1 + b1[None, :]
        g2f = jnp.dot(h1[:, :d_hid], wf_ref[...], preferred_element_type=jnp.float32)
        g2b = jnp.dot(h1[:, d_hid:], wb_ref[...], preferred_element_type=jnp.float32)
        o0[...] = g2f * dinv
        o1[...] = g2b * dinv

    row_spec = pl.BlockSpec((RB, CW), lambda i: (i, 0))
    vec_spec = pl.BlockSpec((d_hid,), lambda i: (0,))
    mat_spec = pl.BlockSpec((d_hid, CW), lambda i: (0, 0))
    return pl.pallas_call(
        body,
        grid=(n // RB,),
        in_specs=[row_spec] * (2 * nch_in + 2) + [vec_spec] * 2 + [mat_spec] * 2,
        out_specs=[row_spec] * 2,
        out_shape=[jax.ShapeDtypeStruct((n, K), jnp.float32) for _ in range(2)],
    )(*s1, *g1, *deg_t, b_f1, b_b1, w_f2, w_b2)


def _tc_final(s2, g2, deg_t, b_f2, b_b2, gamma, beta, w_lin, b_lin, n, d_out):
    def body(s0, s1, g0, g1, deg0, deg1, bf_ref, bb_ref, gam_ref, bet_ref,
             wl_ref, bl_ref, out_ref):
        dinv = _dinv_col((deg0, deg1), n)
        b2 = jnp.concatenate([bf_ref[...], bb_ref[...]])
        h2 = jnp.concatenate(
            [s0[0:n, :] + g0[...], s1[0:n, :] + g1[...]], axis=1
        )
        h2 = dinv * h2 + b2[None, :]
        mean = jnp.mean(h2, axis=0, keepdims=True)
        cen = h2 - mean
        var = jnp.mean(cen * cen, axis=0, keepdims=True)
        hn = cen * lax.rsqrt(var + 1e-5)
        hn = hn * gam_ref[...][None, :] + bet_ref[...][None, :]
        out_ref[...] = (
            jnp.dot(hn, wl_ref[...], preferred_element_type=jnp.float32)
            + bl_ref[...][None, :]
        )

    return pl.pallas_call(
        body,
        out_shape=jax.ShapeDtypeStruct((n, d_out), jnp.float32),
    )(*s2, *g2, *deg_t, b_f2, b_b2, gamma, beta, w_lin, b_lin)


def kernel(x, edge_index, W_f1, b_f1, W_f2, b_f2, W_b1, b_b1, W_b2, b_b2,
           gamma, beta, W_lin, b_lin):
    n, _ = x.shape
    d_hid = W_f1.shape[1]
    d_out = W_f2.shape[1]
    e = edge_index.shape[1]

    ept = NS * EB
    nblk = ((e + ept - 1) // ept + 39) // 40 * 40
    epad = nblk * ept
    npad = (n // (NS * 8) + 2) * (NS * 8)

    src = edge_index[0].astype(jnp.int32)
    dst = edge_index[1].astype(jnp.int32)
    pad = epad - e
    arp = jnp.arange(pad, dtype=jnp.int32)
    trash = n + arp % (npad - n)
    src3 = jnp.concatenate([src, arp % n])
    dst3 = jnp.concatenate([dst, trash])
    src3 = src3.reshape(NS, nblk, EB)
    dst3 = dst3.reshape(NS, nblk, EB)

    ones_h = jnp.ones((EB, DEGW), jnp.float32)
    zeros16 = jnp.zeros((npad // NS, DEGW), jnp.float32)
    zerosK = jnp.zeros((npad // NS, CW), jnp.float32)

    deg_t = _sc_degree(dst3, ones_h, zeros16, npad, nblk)
    g1 = _tc_prep(x, W_f1, W_b1, deg_t, n, d_hid)
    s1 = _sc_spmm(g1, src3, dst3, zerosK, npad, nblk)
    g2 = _tc_mid(s1, g1, deg_t, b_f1, b_b1, W_f2, W_b2, n, d_hid)
    s2 = _sc_spmm(g2, src3, dst3, zerosK, npad, nblk)
    return _tc_final(s2, g2, deg_t, b_f2, b_b2, gamma, beta, W_lin, b_lin, n, d_out)
